# baseline (speedup 1.0000x reference)
import functools

import jax
import jax.numpy as jnp
from jax import lax
from jax.experimental import pallas as pl
from jax.experimental.pallas import tpu as pltpu
from jax.experimental.pallas import tpu_sc as plsc

B = 64
D_MODEL = 1024
D_FF = 2048
E = 16
BF = 1024
NF = D_FF // BF

NC = 2
NS = 16
NW = NC * NS
TPW = B // NW


def _logits_body(x_ref, rw_ref, out_ref):
    out_ref[...] = lax.dot_general(
        x_ref[...], rw_ref[...], (((1,), (1,)), ((), ())),
        preferred_element_type=jnp.float32,
        precision=lax.Precision.HIGHEST)


def _route_sc_body(logits_hbm, coeff_hbm, imp_hbm, load_hbm,
                   lg_v, cf_v, st_v):
    wid = lax.axis_index("s") * NC + lax.axis_index("c")
    base = wid * TPW
    pltpu.sync_copy(logits_hbm.at[pl.ds(base, TPW)], lg_v)
    iota = lax.broadcasted_iota(jnp.int32, (E,), 0)
    imp = jnp.zeros((E,), jnp.float32)
    ld = jnp.zeros((E,), jnp.float32)
    for t in range(TPW):
        row = lg_v[t, :]
        m = jnp.max(row)
        p = jnp.exp(row - m)
        p = p / jnp.sum(p)
        m1 = jnp.max(p)
        i1 = jnp.min(jnp.where(p == m1, iota, E))
        oneh1 = iota == i1
        p2 = jnp.where(oneh1, -jnp.inf, p)
        m2 = jnp.max(p2)
        i2 = jnp.min(jnp.where(p2 == m2, iota, E))
        oneh2 = iota == i2
        s = m1 + m2
        top2 = jnp.where(oneh1, m1, 0.0) + jnp.where(oneh2, m2, 0.0)
        cf_v[t, :] = top2 / s
        imp = imp + p
        ld = ld + oneh1.astype(jnp.float32)
    st_v[0, :] = imp
    st_v[1, :] = ld
    pltpu.sync_copy(cf_v, coeff_hbm.at[pl.ds(base, TPW)])
    pltpu.sync_copy(st_v.at[pl.ds(0, 1)], imp_hbm.at[pl.ds(wid, 1)])
    pltpu.sync_copy(st_v.at[pl.ds(1, 1)], load_hbm.at[pl.ds(wid, 1)])


def _ffn_body(x_ref, coeff_ref, imp_ref, load_ref,
              w1_ref, w3_ref, w2_ref, out_ref, aux_ref):
    e = pl.program_id(0)
    f = pl.program_id(1)

    @pl.when((e == 0) & (f == 0))
    def _aux():
        imp = jnp.sum(imp_ref[...], axis=0) / B
        ld = jnp.sum(load_ref[...], axis=0) / B
        aux_ref[0] = E * jnp.sum(imp * ld)
        out_ref[...] = jnp.zeros_like(out_ref)

    x = x_ref[...]
    g = jnp.dot(x, w1_ref[0], preferred_element_type=jnp.float32)
    u = jnp.dot(x, w3_ref[0], preferred_element_type=jnp.float32)
    h = (g * jax.nn.sigmoid(g)) * u
    iota = lax.broadcasted_iota(jnp.int32, (B, E), 1)
    ce = jnp.sum(jnp.where(iota == e, coeff_ref[...], 0.0), axis=1,
                 keepdims=True)
    out_ref[...] += jnp.dot(h * ce, w2_ref[0],
                            preferred_element_type=jnp.float32)


def kernel(x, router_w, w1, w2, w3):
    b, s, d = x.shape
    x_flat = x.reshape(-1, d)

    logits = pl.pallas_call(
        _logits_body,
        in_specs=[pl.BlockSpec((B, D_MODEL), lambda: (0, 0)),
                  pl.BlockSpec((E, D_MODEL), lambda: (0, 0))],
        out_specs=pl.BlockSpec((B, E), lambda: (0, 0)),
        out_shape=jax.ShapeDtypeStruct((B, E), jnp.float32),
    )(x_flat, router_w)

    route = pl.kernel(
        _route_sc_body,
        out_type=[
            jax.ShapeDtypeStruct((B, E), jnp.float32),
            jax.ShapeDtypeStruct((NW, E), jnp.float32),
            jax.ShapeDtypeStruct((NW, E), jnp.float32),
        ],
        scratch_types=[
            pltpu.VMEM((TPW, E), jnp.float32),
            pltpu.VMEM((TPW, E), jnp.float32),
            pltpu.VMEM((2, E), jnp.float32),
        ],
        mesh=plsc.VectorSubcoreMesh(core_axis_name="c",
                                    subcore_axis_name="s"),
        compiler_params=pltpu.CompilerParams(needs_layout_passes=False,
                                             skip_device_barrier=True),
    )
    coeff, imp, load = route(logits)

    out, aux = pl.pallas_call(
        _ffn_body,
        grid=(E, NF),
        in_specs=[
            pl.BlockSpec((B, D_MODEL), lambda e, f: (0, 0)),
            pl.BlockSpec((B, E), lambda e, f: (0, 0)),
            pl.BlockSpec((NW, E), lambda e, f: (0, 0)),
            pl.BlockSpec((NW, E), lambda e, f: (0, 0)),
            pl.BlockSpec((1, D_MODEL, BF), lambda e, f: (e, 0, f)),
            pl.BlockSpec((1, D_MODEL, BF), lambda e, f: (e, 0, f)),
            pl.BlockSpec((1, BF, D_MODEL), lambda e, f: (e, f, 0)),
        ],
        out_specs=[
            pl.BlockSpec((B, D_MODEL), lambda e, f: (0, 0)),
            pl.BlockSpec(memory_space=pltpu.SMEM, block_shape=(1,),
                         index_map=lambda e, f: (0,)),
        ],
        out_shape=[
            jax.ShapeDtypeStruct((B, D_MODEL), jnp.float32),
            jax.ShapeDtypeStruct((1,), jnp.float32),
        ],
    )(x_flat, coeff, imp, load, w1, w3, w2)
    return out.reshape(b, s, d), aux[0]


# speedup vs baseline: 1.0183x; 1.0183x over previous
import jax
import jax.numpy as jnp
from jax import lax
from jax.experimental import pallas as pl
from jax.experimental.pallas import tpu as pltpu
from jax.experimental.pallas import tpu_sc as plsc

B = 64
D_MODEL = 1024
D_FF = 2048
E = 16
BF = 1024
NF = D_FF // BF

NC = 2
NS = 16
NW = NC * NS
TPW = B // NW


def _logits_body(x_ref, rw_ref, out_ref):
    out_ref[...] = lax.dot_general(
        x_ref[:, 0, :], rw_ref[...], (((1,), (1,)), ((), ())),
        preferred_element_type=jnp.float32,
        precision=lax.Precision.HIGHEST)


def _route_sc_body(logits_hbm, coeff_hbm, imp_hbm, load_hbm,
                   lg_v, cf_v, st_v, sem):
    wid = lax.axis_index("s") * NC + lax.axis_index("c")
    base = wid * TPW
    pltpu.sync_copy(logits_hbm.at[pl.ds(base, TPW)], lg_v)
    iota = lax.broadcasted_iota(jnp.int32, (E,), 0)
    imp = jnp.zeros((E,), jnp.float32)
    ld = jnp.zeros((E,), jnp.float32)
    for t in range(TPW):
        row = lg_v[t, :]
        m = jnp.max(row)
        p = jnp.exp(row - m)
        p = p / jnp.sum(p)
        m1 = jnp.max(p)
        i1 = jnp.min(jnp.where(p == m1, iota, E))
        oneh1 = iota == i1
        p2 = jnp.where(oneh1, -jnp.inf, p)
        m2 = jnp.max(p2)
        i2 = jnp.min(jnp.where(p2 == m2, iota, E))
        oneh2 = iota == i2
        s = m1 + m2
        top2 = jnp.where(oneh1, m1, 0.0) + jnp.where(oneh2, m2, 0.0)
        cf_v[t, :] = top2 / s
        imp = imp + p
        ld = ld + oneh1.astype(jnp.float32)
    st_v[0, :] = imp
    st_v[1, :] = ld
    c1 = pltpu.async_copy(cf_v, coeff_hbm.at[pl.ds(base, TPW)], sem)
    c2 = pltpu.async_copy(st_v.at[pl.ds(0, 1)],
                          imp_hbm.at[pl.ds(wid, 1)], sem)
    c3 = pltpu.async_copy(st_v.at[pl.ds(1, 1)],
                          load_hbm.at[pl.ds(wid, 1)], sem)
    c1.wait()
    c2.wait()
    c3.wait()


def _ffn_body(x_ref, coeff_ref, imp_ref, load_ref,
              w1_ref, w3_ref, w2_ref, out_ref, aux_ref,
              xs_ref, acc_ref):
    e = pl.program_id(0)
    f = pl.program_id(1)

    @pl.when((e == 0) & (f == 0))
    def _first():
        xs_ref[...] = x_ref[:, 0, :]
        imp = jnp.sum(imp_ref[...], axis=0) / B
        ld = jnp.sum(load_ref[...], axis=0) / B
        aux_ref[0] = E * jnp.sum(imp * ld)
        acc_ref[...] = jnp.zeros_like(acc_ref)

    x = xs_ref[...]
    g = jnp.dot(x, w1_ref[0], preferred_element_type=jnp.float32)
    u = jnp.dot(x, w3_ref[0], preferred_element_type=jnp.float32)
    h = (g * jax.nn.sigmoid(g)) * u
    iota = lax.broadcasted_iota(jnp.int32, (B, E), 1)
    ce = jnp.sum(jnp.where(iota == e, coeff_ref[...], 0.0), axis=1,
                 keepdims=True)
    acc_ref[...] += jnp.dot(h * ce, w2_ref[0],
                            preferred_element_type=jnp.float32)

    @pl.when((e == E - 1) & (f == NF - 1))
    def _last():
        out_ref[:, 0, :] = acc_ref[...]


def kernel(x, router_w, w1, w2, w3):
    b, s, d = x.shape

    logits = pl.pallas_call(
        _logits_body,
        in_specs=[pl.BlockSpec((B, 1, D_MODEL), lambda: (0, 0, 0)),
                  pl.BlockSpec((E, D_MODEL), lambda: (0, 0))],
        out_specs=pl.BlockSpec((B, E), lambda: (0, 0)),
        out_shape=jax.ShapeDtypeStruct((B, E), jnp.float32),
    )(x, router_w)

    route = pl.kernel(
        _route_sc_body,
        out_type=[
            jax.ShapeDtypeStruct((B, E), jnp.float32),
            jax.ShapeDtypeStruct((NW, E), jnp.float32),
            jax.ShapeDtypeStruct((NW, E), jnp.float32),
        ],
        scratch_types=[
            pltpu.VMEM((TPW, E), jnp.float32),
            pltpu.VMEM((TPW, E), jnp.float32),
            pltpu.VMEM((2, E), jnp.float32),
            pltpu.SemaphoreType.DMA,
        ],
        mesh=plsc.VectorSubcoreMesh(core_axis_name="c",
                                    subcore_axis_name="s"),
        compiler_params=pltpu.CompilerParams(needs_layout_passes=False),
    )
    coeff, imp, load = route(logits)

    out, aux = pl.pallas_call(
        _ffn_body,
        grid=(E, NF),
        in_specs=[
            pl.BlockSpec((B, 1, D_MODEL), lambda e, f: (0, 0, 0)),
            pl.BlockSpec((B, E), lambda e, f: (0, 0)),
            pl.BlockSpec((NW, E), lambda e, f: (0, 0)),
            pl.BlockSpec((NW, E), lambda e, f: (0, 0)),
            pl.BlockSpec((1, D_MODEL, BF), lambda e, f: (e, 0, f)),
            pl.BlockSpec((1, D_MODEL, BF), lambda e, f: (e, 0, f)),
            pl.BlockSpec((1, BF, D_MODEL), lambda e, f: (e, f, 0)),
        ],
        out_specs=[
            pl.BlockSpec((B, 1, D_MODEL), lambda e, f: (0, 0, 0)),
            pl.BlockSpec(memory_space=pltpu.SMEM, block_shape=(1,),
                         index_map=lambda e, f: (0,)),
        ],
        out_shape=[
            jax.ShapeDtypeStruct((B, 1, D_MODEL), jnp.float32),
            jax.ShapeDtypeStruct((1,), jnp.float32),
        ],
        scratch_shapes=[pltpu.VMEM((B, D_MODEL), jnp.float32),
                        pltpu.VMEM((B, D_MODEL), jnp.float32)],
    )(x, coeff, imp, load, w1, w3, w2)
    return out, aux[0]
